# baseline (speedup 1.0000x reference)
import jax
import jax.numpy as jnp
from jax.experimental import pallas as pl
from jax.experimental.pallas import tpu as pltpu

_S = 64.0
_M = 0.4

_R = 400
_NBUF = 8


def _body(lbl_hbm, cos_hbm, out_hbm, lblbuf, inbufs, outbufs, lblsem, insems, outsems):
    i = pl.program_id(0)
    nchunk = pl.num_programs(0)
    slot = jax.lax.rem(i, _NBUF)

    @pl.when(i == 0)
    def _prologue():
        for k in range(_NBUF):
            pltpu.make_async_copy(
                cos_hbm.at[pl.ds(k * _R, _R)], inbufs.at[k], insems.at[k]
            ).start()
        pltpu.make_async_copy(lbl_hbm, lblbuf, lblsem).start()

    pltpu.make_async_copy(
        cos_hbm.at[pl.ds(i * _R, _R)], inbufs.at[slot], insems.at[slot]
    ).wait()

    @pl.when(i == 0)
    def _wait_label():
        pltpu.make_async_copy(lbl_hbm, lblbuf, lblsem).wait()

    @pl.when(i >= _NBUF)
    def _drain_prev():
        pltpu.make_async_copy(
            outbufs.at[slot],
            out_hbm.at[pl.ds((i - _NBUF) * _R, _R)],
            outsems.at[slot],
        ).wait()

    classes = jax.lax.broadcasted_iota(jnp.int32, (_R, cos_hbm.shape[1]), 0) + i * _R
    hit = lblbuf[...] == classes
    outbufs[slot] = inbufs[slot] * _S - jnp.where(hit, _M * _S, 0.0)

    pltpu.make_async_copy(
        outbufs.at[slot], out_hbm.at[pl.ds(i * _R, _R)], outsems.at[slot]
    ).start()

    @pl.when(i + _NBUF < nchunk)
    def _next_in():
        pltpu.make_async_copy(
            cos_hbm.at[pl.ds((i + _NBUF) * _R, _R)], inbufs.at[slot], insems.at[slot]
        ).start()

    @pl.when(i == nchunk - 1)
    def _epilogue():
        for j in range(_NBUF):
            s = nchunk - _NBUF + j
            pltpu.make_async_copy(
                outbufs.at[s % _NBUF],
                out_hbm.at[pl.ds(s * _R, _R)],
                outsems.at[s % _NBUF],
            ).wait()


def kernel(cosine, label):
    batch, num_classes = cosine.shape
    cos_t = cosine.T
    lbl2d = label.astype(jnp.int32).reshape(1, batch)
    nchunk = num_classes // _R
    out_t = pl.pallas_call(
        _body,
        grid=(nchunk,),
        in_specs=[
            pl.BlockSpec(memory_space=pl.ANY),
            pl.BlockSpec(memory_space=pl.ANY),
        ],
        out_specs=pl.BlockSpec(memory_space=pl.ANY),
        out_shape=jax.ShapeDtypeStruct((num_classes, batch), cosine.dtype),
        scratch_shapes=[
            pltpu.VMEM((1, batch), jnp.int32),
            pltpu.VMEM((_NBUF, _R, batch), cosine.dtype),
            pltpu.VMEM((_NBUF, _R, batch), cosine.dtype),
            pltpu.SemaphoreType.DMA,
            pltpu.SemaphoreType.DMA((_NBUF,)),
            pltpu.SemaphoreType.DMA((_NBUF,)),
        ],
    )(lbl2d, cos_t)
    return out_t.T


# speedup vs baseline: 1.0015x; 1.0015x over previous
import jax
import jax.numpy as jnp
from jax.experimental import pallas as pl
from jax.experimental.pallas import tpu as pltpu

_S = 64.0
_M = 0.4

_R = 400
_NBUF = 8
_RH = 96


def _body(lbl_ref, cos_hbm, out_hbm, inbufs, outbufs, insems, outsems):
    i = pl.program_id(0)
    nchunk = pl.num_programs(0)
    slot = jax.lax.rem(i, _NBUF)
    lbl = lbl_ref[...]

    def _compute(dst, src, row0, nrows):
        classes = (
            jax.lax.broadcasted_iota(jnp.int32, (nrows, lbl.shape[1]), 0) + row0
        )
        hit = lbl == classes
        dst[...] = src[...] * _S - jnp.where(hit, _M * _S, 0.0)

    @pl.when(i == 0)
    def _first_chunk():
        head = pltpu.make_async_copy(
            cos_hbm.at[pl.ds(0, _RH)], inbufs.at[0, pl.ds(0, _RH)], outsems.at[0]
        )
        rest = pltpu.make_async_copy(
            cos_hbm.at[pl.ds(_RH, _R - _RH)],
            inbufs.at[0, pl.ds(_RH, _R - _RH)],
            insems.at[0],
        )
        head.start()
        rest.start()
        for k in range(1, _NBUF):
            pltpu.make_async_copy(
                cos_hbm.at[pl.ds(k * _R, _R)], inbufs.at[k], insems.at[k]
            ).start()
        head.wait()
        _compute(outbufs.at[0, pl.ds(0, _RH)], inbufs.at[0, pl.ds(0, _RH)], 0, _RH)
        pltpu.make_async_copy(
            outbufs.at[0, pl.ds(0, _RH)], out_hbm.at[pl.ds(0, _RH)], outsems.at[0]
        ).start()
        rest.wait()
        _compute(
            outbufs.at[0, pl.ds(_RH, _R - _RH)],
            inbufs.at[0, pl.ds(_RH, _R - _RH)],
            _RH,
            _R - _RH,
        )
        pltpu.make_async_copy(
            outbufs.at[0, pl.ds(_RH, _R - _RH)],
            out_hbm.at[pl.ds(_RH, _R - _RH)],
            outsems.at[0],
        ).start()

    @pl.when(i > 0)
    def _steady():
        pltpu.make_async_copy(
            cos_hbm.at[pl.ds(i * _R, _R)], inbufs.at[slot], insems.at[slot]
        ).wait()

        @pl.when(i >= _NBUF)
        def _drain_prev():
            pltpu.make_async_copy(
                outbufs.at[slot],
                out_hbm.at[pl.ds((i - _NBUF) * _R, _R)],
                outsems.at[slot],
            ).wait()

        _compute(outbufs.at[slot], inbufs.at[slot], i * _R, _R)
        pltpu.make_async_copy(
            outbufs.at[slot], out_hbm.at[pl.ds(i * _R, _R)], outsems.at[slot]
        ).start()

    @pl.when(i + _NBUF < nchunk)
    def _next_in():
        pltpu.make_async_copy(
            cos_hbm.at[pl.ds((i + _NBUF) * _R, _R)], inbufs.at[slot], insems.at[slot]
        ).start()

    @pl.when(i == nchunk - 1)
    def _epilogue():
        for j in range(_NBUF):
            s = nchunk - _NBUF + j
            pltpu.make_async_copy(
                outbufs.at[s % _NBUF],
                out_hbm.at[pl.ds(s * _R, _R)],
                outsems.at[s % _NBUF],
            ).wait()


def kernel(cosine, label):
    batch, num_classes = cosine.shape
    cos_t = cosine.T
    lbl2d = label.astype(jnp.int32).reshape(1, batch)
    nchunk = num_classes // _R
    out_t = pl.pallas_call(
        _body,
        grid=(nchunk,),
        in_specs=[
            pl.BlockSpec(memory_space=pltpu.VMEM),
            pl.BlockSpec(memory_space=pl.ANY),
        ],
        out_specs=pl.BlockSpec(memory_space=pl.ANY),
        out_shape=jax.ShapeDtypeStruct((num_classes, batch), cosine.dtype),
        scratch_shapes=[
            pltpu.VMEM((_NBUF, _R, batch), cosine.dtype),
            pltpu.VMEM((_NBUF, _R, batch), cosine.dtype),
            pltpu.SemaphoreType.DMA((_NBUF,)),
            pltpu.SemaphoreType.DMA((_NBUF,)),
        ],
    )(lbl2d, cos_t)
    return out_t.T
